# Initial kernel scaffold; baseline (speedup 1.0000x reference)
import jax
import jax.numpy as jnp
from jax.experimental import pallas as pl


def kernel(x, edge_index, batch, params):
    raise NotImplementedError("write your pallas kernel here")


# speedup vs baseline: 1.0386x; 1.0386x over previous
import functools

import jax
import jax.numpy as jnp
from jax import lax
from jax.experimental import pallas as pl
from jax.experimental.pallas import tpu as pltpu
from jax.experimental.pallas import tpu_sc as plsc

NN = 10000
NP = 10240
NE = 160000
EP = 163840
EB = 2048
NEB = EP // EB
NG = 100
BN_EPS = 1e-5

K = 128
NC, NS = 2, 16
NW = NC * NS
EPW = EP // NW
GCH = EPW // K
EPS_SC = EP // NS
SCH = EPS_SC // K
HALF = NP // 2
SROWS = HALF + 128
GARB = HALF
PADIDX = NN + 8

NTB = 10
NB = NP // NTB
CW = 128

@functools.lru_cache(maxsize=None)
def _get_mesh():
    return plsc.VectorSubcoreMesh(
        core_axis_name="c", subcore_axis_name="s",
        num_cores=NC, num_subcores=NS)


@functools.lru_cache(maxsize=None)
def _make_gather(F):
    @functools.partial(
        pl.kernel,
        out_type=[jax.ShapeDtypeStruct((EP, F), jnp.float32),
                  jax.ShapeDtypeStruct((EP, F), jnp.float32)],
        mesh=_get_mesh(),
        scratch_types=[pltpu.VMEM((K,), jnp.int32),
                       pltpu.VMEM((K, F), jnp.float32),
                       pltpu.VMEM((K,), jnp.int32),
                       pltpu.VMEM((K, F), jnp.float32),
                       pltpu.SemaphoreType.DMA,
                       pltpu.SemaphoreType.DMA],
    )
    def gather(p_hbm, q_hbm, dst_hbm, src_hbm, tp_hbm, tq_hbm,
               idx_d, rows_d, idx_s, rows_s, sem_d, sem_s):
        wid = lax.axis_index("s") * NC + lax.axis_index("c")
        base = wid * EPW

        def body(j, carry):
            off = base + j * K
            pltpu.sync_copy(dst_hbm.at[pl.ds(off, K)], idx_d)
            pltpu.async_copy(p_hbm.at[idx_d], rows_d, sem_d).wait()
            pltpu.sync_copy(rows_d, tp_hbm.at[pl.ds(off, K)])
            pltpu.sync_copy(src_hbm.at[pl.ds(off, K)], idx_s)
            pltpu.async_copy(q_hbm.at[idx_s], rows_s, sem_s).wait()
            pltpu.sync_copy(rows_s, tq_hbm.at[pl.ds(off, K)])
            return carry

        lax.fori_loop(0, GCH, body, 0)

    return gather


def _segsum(z, dstp, with_counts):
    s = jax.ops.segment_sum(z, dstp, num_segments=NP)
    c = None
    if with_counts:
        c = jax.ops.segment_sum(jnp.ones((EP,), jnp.float32), dstp,
                                num_segments=NP)
    return s, c


def _node_proj_first_body(x_ref, wp_ref, wq_ref, p_ref, q_ref):
    xn = x_ref[...]
    p_ref[...] = jnp.dot(xn, wp_ref[...], preferred_element_type=jnp.float32)
    q_ref[...] = jnp.dot(xn, wq_ref[...], preferred_element_type=jnp.float32)


def _node_proj_body(s_ref, cnt_ref, wp_ref, wq_ref, p_ref, q_ref):
    xn = jnp.maximum(s_ref[...] / jnp.maximum(cnt_ref[...], 1.0), 0.0)
    p_ref[...] = jnp.dot(xn, wp_ref[...], preferred_element_type=jnp.float32)
    q_ref[...] = jnp.dot(xn, wq_ref[...], preferred_element_type=jnp.float32)


def _node_proj_first(xp, wp, wq):
    F = wp.shape[1]
    return pl.pallas_call(
        _node_proj_first_body,
        out_shape=[jax.ShapeDtypeStruct((NP, F), jnp.float32),
                   jax.ShapeDtypeStruct((NP, F), jnp.float32)],
    )(xp, wp, wq)


def _node_proj(s, cnt, wp, wq):
    F = wp.shape[1]
    return pl.pallas_call(
        _node_proj_body,
        out_shape=[jax.ShapeDtypeStruct((NP, F), jnp.float32),
                   jax.ShapeDtypeStruct((NP, F), jnp.float32)],
    )(s, cnt, wp, wq)


def _pass0_body(tp_ref, tq_ref, acc_ref):
    i = pl.program_id(0)
    t = tp_ref[...] + tq_ref[...]

    @pl.when(i == 0)
    def _():
        acc_ref[...] = jnp.zeros_like(acc_ref)

    acc_ref[0:1, :] += jnp.sum(t, axis=0, keepdims=True)
    acc_ref[1:2, :] += jnp.sum(t * t, axis=0, keepdims=True)


def _pass0(tp, tq):
    F = tp.shape[1]
    return pl.pallas_call(
        _pass0_body,
        grid=(NEB,),
        in_specs=[pl.BlockSpec((EB, F), lambda i: (i, 0)),
                  pl.BlockSpec((EB, F), lambda i: (i, 0))],
        out_specs=pl.BlockSpec((8, F), lambda i: (0, 0)),
        out_shape=jax.ShapeDtypeStruct((8, F), jnp.float32),
    )(tp, tq)


def _mask_rows(z, i):
    rowid = lax.broadcasted_iota(jnp.int32, z.shape, 0) + i * EB
    return jnp.where(rowid < NE, z, 0.0)


def _passmm_two_body(tp_ref, tq_ref, ac_ref, w_ref, u_ref, acc_ref):
    i = pl.program_id(0)
    t = tp_ref[...] + tq_ref[...]
    z = jnp.maximum(t * ac_ref[0:1, :] + ac_ref[1:2, :], 0.0)
    z = _mask_rows(z, i)
    u = jnp.dot(z, w_ref[...], preferred_element_type=jnp.float32)
    u_ref[...] = u

    @pl.when(i == 0)
    def _():
        acc_ref[...] = jnp.zeros_like(acc_ref)

    acc_ref[0:1, :] += jnp.sum(u, axis=0, keepdims=True)
    acc_ref[1:2, :] += jnp.sum(u * u, axis=0, keepdims=True)


def _passmm_one_body(t_ref, ac_ref, w_ref, u_ref, acc_ref):
    i = pl.program_id(0)
    z = jnp.maximum(t_ref[...] * ac_ref[0:1, :] + ac_ref[1:2, :], 0.0)
    z = _mask_rows(z, i)
    u = jnp.dot(z, w_ref[...], preferred_element_type=jnp.float32)
    u_ref[...] = u

    @pl.when(i == 0)
    def _():
        acc_ref[...] = jnp.zeros_like(acc_ref)

    acc_ref[0:1, :] += jnp.sum(u, axis=0, keepdims=True)
    acc_ref[1:2, :] += jnp.sum(u * u, axis=0, keepdims=True)


def _pass_two(tp, tq, ac, w):
    F = tp.shape[1]
    Fo = w.shape[1]
    return pl.pallas_call(
        _passmm_two_body,
        grid=(NEB,),
        in_specs=[pl.BlockSpec((EB, F), lambda i: (i, 0)),
                  pl.BlockSpec((EB, F), lambda i: (i, 0)),
                  pl.BlockSpec((8, F), lambda i: (0, 0)),
                  pl.BlockSpec((F, Fo), lambda i: (0, 0))],
        out_specs=[pl.BlockSpec((EB, Fo), lambda i: (i, 0)),
                   pl.BlockSpec((8, Fo), lambda i: (0, 0))],
        out_shape=[jax.ShapeDtypeStruct((EP, Fo), jnp.float32),
                   jax.ShapeDtypeStruct((8, Fo), jnp.float32)],
    )(tp, tq, ac, w)


def _pass_one(t, ac, w):
    F = t.shape[1]
    Fo = w.shape[1]
    return pl.pallas_call(
        _passmm_one_body,
        grid=(NEB,),
        in_specs=[pl.BlockSpec((EB, F), lambda i: (i, 0)),
                  pl.BlockSpec((8, F), lambda i: (0, 0)),
                  pl.BlockSpec((F, Fo), lambda i: (0, 0))],
        out_specs=[pl.BlockSpec((EB, Fo), lambda i: (i, 0)),
                   pl.BlockSpec((8, Fo), lambda i: (0, 0))],
        out_shape=[jax.ShapeDtypeStruct((EP, Fo), jnp.float32),
                   jax.ShapeDtypeStruct((8, Fo), jnp.float32)],
    )(t, ac, w)


def _pass3_body(t_ref, ac_ref, z_ref):
    i = pl.program_id(0)
    z = jnp.maximum(t_ref[...] * ac_ref[0:1, :] + ac_ref[1:2, :], 0.0)
    z_ref[...] = _mask_rows(z, i)


def _pass3(t, ac):
    F = t.shape[1]
    return pl.pallas_call(
        _pass3_body,
        grid=(NEB,),
        in_specs=[pl.BlockSpec((EB, F), lambda i: (i, 0)),
                  pl.BlockSpec((8, F), lambda i: (0, 0))],
        out_specs=pl.BlockSpec((EB, F), lambda i: (i, 0)),
        out_shape=jax.ShapeDtypeStruct((EP, F), jnp.float32),
    )(t, ac)


def _tail_body(s_ref, cnt_ref, b_ref, w1_ref, b1_ref, w2_ref, b2_ref,
               o_ref, pool_ref):
    i = pl.program_id(0)

    @pl.when(i == 0)
    def _():
        pool_ref[...] = jnp.full_like(pool_ref, -1.0)
        o_ref[...] = jnp.zeros_like(o_ref)

    h = jnp.maximum(s_ref[...] / jnp.maximum(cnt_ref[...], 1.0), 0.0)
    bt = b_ref[0, 0, :]
    lo = b_ref[0, 0, 0]
    hi = b_ref[0, 0, NB - 1]

    def gbody(g, carry):
        m = jnp.max(jnp.where(bt[:, None] == g, h, -1.0), axis=0)
        cur = pool_ref[pl.ds(g, 1), :]
        pool_ref[pl.ds(g, 1), :] = jnp.maximum(cur, m[None, :])
        return carry

    lax.fori_loop(lo, hi + 1, gbody, 0)

    @pl.when(i == NTB - 1)
    def _():
        pooled = jnp.maximum(pool_ref[...], 0.0)
        h1 = jnp.maximum(
            jnp.dot(pooled, w1_ref[...], preferred_element_type=jnp.float32)
            + b1_ref[...], 0.0)
        o_ref[...] = (jnp.dot(h1, w2_ref[...],
                              preferred_element_type=jnp.float32)
                      + b2_ref[...])


def _tail(s3, cnt, batch3, w1, b1, w2, b2):
    return pl.pallas_call(
        _tail_body,
        grid=(NTB,),
        in_specs=[pl.BlockSpec((NB, 256), lambda i: (i, 0)),
                  pl.BlockSpec((NB, 1), lambda i: (i, 0)),
                  pl.BlockSpec((1, 1, NB), lambda i: (i, 0, 0)),
                  pl.BlockSpec((256, 256), lambda i: (0, 0)),
                  pl.BlockSpec((1, 256), lambda i: (0, 0)),
                  pl.BlockSpec((256, 2), lambda i: (0, 0)),
                  pl.BlockSpec((1, 2), lambda i: (0, 0))],
        out_specs=pl.BlockSpec((128, 2), lambda i: (0, 0)),
        out_shape=jax.ShapeDtypeStruct((128, 2), jnp.float32),
        scratch_shapes=[pltpu.VMEM((128, 256), jnp.float32)],
    )(s3, cnt, batch3, w1, b1, w2, b2)


def _affine(sums, gamma, beta):
    mu = sums[0] / NE
    var = sums[1] / NE - mu * mu
    a = gamma / jnp.sqrt(var + BN_EPS)
    c = beta - mu * a
    pad = jnp.zeros((6, a.shape[0]), jnp.float32)
    return jnp.concatenate([a[None, :], c[None, :], pad], axis=0)


def _gather_call(F, pn, qn, dstp, srcp):
    return _make_gather(F)(pn, qn, dstp, srcp)


def _conv(pn, qn, dstp, srcp, layers, with_counts):
    F0 = pn.shape[1]
    tp, tq = _gather_call(F0, pn, qn, dstp, srcp)
    s0 = _pass0(tp, tq)
    ac1 = _affine(s0, layers[0]["gamma"], layers[0]["beta"])
    u2, s1 = _pass_two(tp, tq, ac1, layers[1]["W"])
    ac2 = _affine(s1, layers[1]["gamma"], layers[1]["beta"])
    u3, s2 = _pass_one(u2, ac2, layers[2]["W"])
    ac3 = _affine(s2, layers[2]["gamma"], layers[2]["beta"])
    z = _pass3(u3, ac3)
    s_out, c_out = _segsum(z, dstp, with_counts)
    if with_counts:
        return s_out, c_out[:, None]
    return s_out, None


def _split_w(w):
    F = w.shape[0] // 2
    return w[:F] - w[F:], w[F:]


def kernel(x, edge_index, batch, params):
    dst = edge_index[1]
    src = edge_index[0]
    pad_i = jnp.full((EP - NE,), PADIDX, jnp.int32)
    dstp = jnp.concatenate([dst, pad_i])
    srcp = jnp.concatenate([src, pad_i])
    xp = jnp.pad(x, ((0, NP - NN), (0, 0)))
    batchp = jnp.concatenate([batch, jnp.full((NP - NN,), 127, jnp.int32)])
    batch3 = batchp.reshape(NTB, 1, NB)

    w1 = params["conv1"][0]["W"]
    wp1 = jnp.pad(w1[:4] - w1[4:], ((0, 0), (0, 64)))
    wq1 = jnp.pad(w1[4:], ((0, 0), (0, 64)))
    lay1 = [{"W": None,
             "gamma": jnp.pad(params["conv1"][0]["gamma"], (0, 64)),
             "beta": jnp.pad(params["conv1"][0]["beta"], (0, 64))}]
    for p in params["conv1"][1:]:
        lay1.append({"W": jnp.pad(p["W"], ((0, 64), (0, 64))),
                     "gamma": jnp.pad(p["gamma"], (0, 64)),
                     "beta": jnp.pad(p["beta"], (0, 64))})
    p1, q1 = _node_proj_first(xp, wp1, wq1)
    s1, cnt = _conv(p1, q1, dstp, srcp, lay1, True)

    w2 = params["conv2"][0]["W"]
    wp2 = jnp.pad(w2[:64] - w2[64:], ((0, 64), (0, 0)))
    wq2 = jnp.pad(w2[64:], ((0, 64), (0, 0)))
    p2, q2 = _node_proj(s1, cnt, wp2, wq2)
    s2, _ = _conv(p2, q2, dstp, srcp, params["conv2"], False)

    wp3, wq3 = _split_w(params["conv3"][0]["W"])
    p3, q3 = _node_proj(s2, cnt, wp3, wq3)
    s3, _ = _conv(p3, q3, dstp, srcp, params["conv3"], False)

    out = _tail(s3, cnt, batch3,
                params["fc1_W"], params["fc1_b"][None, :],
                params["out_W"], params["out_b"][None, :])
    return out[:NG]
